# Initial kernel scaffold; baseline (speedup 1.0000x reference)
import jax
import jax.numpy as jnp
from jax.experimental import pallas as pl


def kernel(vertices, body_pose, left_hand_pose, right_hand_pose, init_pose, init_verts, geodist, hand_contact_prior_weights, ds, hand_contact_prior, faces, init_verts_in_contact):
    raise NotImplementedError("write your pallas kernel here")


# speedup vs baseline: 2.8366x; 2.8366x over previous
import functools
import jax
import jax.numpy as jnp
from jax import lax
from jax.experimental import pallas as pl
from jax.experimental.pallas import tpu as pltpu

_NV = 6890
_GEO_THRESH = 0.1
_A1, _A2, _B1, _B2 = 0.04, 0.04, 0.07, 0.06
_C1, _C2, _D1, _D2 = 0.01, 0.01, 0.023, 0.02
_INSIDE_W, _OUTSIDE_W, _CONTACT_W = 0.5, 2.0, 0.5
_HCP_W, _POSE_W, _HAND_POSE_W, _ANGLE_W = 1.0, 0.04, 0.04, 0.01

_R = 256
_NBLK = (_NV + _R - 1) // _R
_NVP = _NBLK * _R


def _stream_body(geo_ref, vr_ref, vt_ref, ivc_ref, minv_ref, mini_ref,
                 gdi_ref, cmask_ref):
    i = pl.program_id(0)

    @pl.when(i == 0)
    def _():
        col = lax.broadcasted_iota(jnp.int32, (1, _NV), 1)
        acc = jnp.zeros((1, _NV), jnp.float32)
        nc = ivc_ref.shape[0]
        chunk = 100
        for j in range(0, nc, chunk):
            ids = ivc_ref[j:j + chunk, :]
            hit = (ids == col).astype(jnp.float32)
            acc = jnp.maximum(acc, jnp.max(hit, axis=0, keepdims=True))
        cmask_ref[...] = acc

    geo = geo_ref[...]
    vr = vr_ref[...]
    vt = vt_ref[...]

    sqr = jnp.sum(vr * vr, axis=1, keepdims=True)
    sqa = jnp.sum(vt * vt, axis=0, keepdims=True)
    prod = jnp.dot(vr, vt, preferred_element_type=jnp.float32)
    d2 = sqr + sqa - 2.0 * prod
    dist = jnp.sqrt(jnp.maximum(d2, 1e-12))
    distm = jnp.where(geo < _GEO_THRESH, 1e5, dist)

    minv = jnp.min(distm, axis=1, keepdims=True)
    coli = lax.broadcasted_iota(jnp.int32, (_R, _NV), 1)
    idx = jnp.min(jnp.where(distm <= minv, coli, _NV), axis=1, keepdims=True)

    cmask = cmask_ref[...]
    gdi = jnp.min(jnp.where(cmask > 0.0, geo, 1e5), axis=1, keepdims=True)

    minv_ref[...] = minv
    mini_ref[...] = idx
    gdi_ref[...] = gdi


@jax.jit
def _stream_minmax(geodist, v, init_verts_in_contact):
    vt = v.T
    vr = jnp.pad(v, ((0, _NVP - _NV), (0, 0)))
    ivc = init_verts_in_contact.reshape(-1, 1).astype(jnp.int32)

    out = pl.pallas_call(
        _stream_body,
        grid=(_NBLK,),
        in_specs=[
            pl.BlockSpec((_R, _NV), lambda i: (i, 0)),
            pl.BlockSpec((_R, 3), lambda i: (i, 0)),
            pl.BlockSpec((3, _NV), lambda i: (0, 0)),
            pl.BlockSpec((ivc.shape[0], 1), lambda i: (0, 0)),
        ],
        out_specs=[
            pl.BlockSpec((_R, 1), lambda i: (i, 0)),
            pl.BlockSpec((_R, 1), lambda i: (i, 0)),
            pl.BlockSpec((_R, 1), lambda i: (i, 0)),
        ],
        out_shape=[
            jax.ShapeDtypeStruct((_NVP, 1), jnp.float32),
            jax.ShapeDtypeStruct((_NVP, 1), jnp.int32),
            jax.ShapeDtypeStruct((_NVP, 1), jnp.float32),
        ],
        scratch_shapes=[pltpu.VMEM((1, _NV), jnp.float32)],
    )(geodist, vr, vt, ivc)
    minv, mini, gdi = out
    return minv[:_NV, 0], mini[:_NV, 0], gdi[:_NV, 0]


def _masked_mean(vals, mask):
    m = mask.astype(vals.dtype)
    cnt = jnp.sum(m)
    return jnp.where(cnt > 0, jnp.sum(vals * m) / jnp.maximum(cnt, 1.0), 0.0)


def kernel(vertices, body_pose, left_hand_pose, right_hand_pose, init_pose,
           init_verts, geodist, hand_contact_prior_weights, ds,
           hand_contact_prior, faces, init_verts_in_contact):
    v = vertices[0]
    nv = v.shape[0]

    v2v_min, v2v_min_idx, gdi = _stream_minmax(geodist, v, init_verts_in_contact)

    v0 = v[faces[:, 0]]
    v1 = v[faces[:, 1]]
    v2 = v[faces[:, 2]]
    fn = jnp.cross(v1 - v0, v2 - v0)
    vn = jnp.zeros_like(v)
    vn = vn.at[faces[:, 0]].add(fn)
    vn = vn.at[faces[:, 1]].add(fn)
    vn = vn.at[faces[:, 2]].add(fn)
    vn = vn / (jnp.linalg.norm(vn, axis=1, keepdims=True) + 1e-8)

    nearest = v2v_min_idx[ds]
    exterior = jnp.sum((v[ds] - v[nearest]) * vn[nearest], axis=-1) > 0
    inside = (jnp.zeros(nv, dtype=jnp.float32).at[ds].add(
        (~exterior).astype(jnp.float32)) > 0)

    weights_outside = 1.0 / (5.0 * gdi + 1.0)
    vals_out = _A1 * weights_outside[ds] * jnp.tanh(v2v_min[ds] / _A2)
    contactloss = _CONTACT_W * _masked_mean(vals_out, ~inside[ds])

    vals_in = _B1 * jnp.tanh(v2v_min / _B2)
    insideloss = _INSIDE_W * _masked_mean(vals_in, inside)

    ha = hand_contact_prior.shape[0] // 2
    hvi = inside[hand_contact_prior]
    lvals_o = _C1 * jnp.tanh(v2v_min[hand_contact_prior[:ha]] / _C2)
    rvals_o = _C1 * jnp.tanh(v2v_min[hand_contact_prior[ha:]] / _C2)
    lw = -0.1 * hand_contact_prior_weights[:ha] + 1.0
    rw = -0.1 * hand_contact_prior_weights[ha:] + 1.0
    hclo = (_masked_mean(lw * lvals_o, ~hvi[:ha]) +
            _masked_mean(rw * rvals_o, ~hvi[ha:]))
    lvals_i = _D1 * jnp.tanh(v2v_min[hand_contact_prior[:ha]] / _D2)
    rvals_i = _D1 * jnp.tanh(v2v_min[hand_contact_prior[ha:]] / _D2)
    hcli = _masked_mean(lvals_i, hvi[:ha]) + _masked_mean(rvals_i, hvi[ha:])
    hand_contact_loss = _HCP_W * (hcli + hclo)

    ng = jnp.sum(vn * vn[v2v_min_idx], axis=-1)
    angle_loss = _ANGLE_W * _masked_mean(1.0 + ng, v2v_min < 0.01)

    pose_prior_loss = _POSE_W * jnp.sum((body_pose - init_pose) ** 2)
    hand_pose_prior_loss = _HAND_POSE_W * (jnp.sum(left_hand_pose ** 2) +
                                           jnp.sum(right_hand_pose ** 2))

    ov = jnp.linalg.norm(init_verts - vertices, axis=2)
    ow = (2.0 * gdi.reshape(vertices.shape[0], -1)) ** 2
    outsideloss = _OUTSIDE_W * jnp.sum(ov * ow)

    return (contactloss + insideloss + outsideloss + pose_prior_loss
            + hand_pose_prior_loss + angle_loss + hand_contact_loss)


# speedup vs baseline: 5.7442x; 2.0250x over previous
import functools
import jax
import jax.numpy as jnp
from jax import lax
from jax.experimental import pallas as pl
from jax.experimental.pallas import tpu as pltpu
from jax.experimental.pallas import tpu_sc as plsc

_NV = 6890
_GEO_THRESH = 0.1
_A1, _A2, _B1, _B2 = 0.04, 0.04, 0.07, 0.06
_C1, _C2, _D1, _D2 = 0.01, 0.01, 0.023, 0.02
_INSIDE_W, _OUTSIDE_W, _CONTACT_W = 0.5, 2.0, 0.5
_HCP_W, _POSE_W, _HAND_POSE_W, _ANGLE_W = 1.0, 0.04, 0.04, 0.01

_R = 256
_NBLK = (_NV + _R - 1) // _R
_NVP = _NBLK * _R

_NF = 13776
_NFP = 14336
_HNF = _NFP // 2
_DS = 1722
_DSP = 1728
_HC = 1556
_HCPAD = 1568


def _stream_body(geo_ref, vr_ref, vt_ref, ivc_ref, minv_ref, mini_ref,
                 gdi_ref, cmask_ref):
    i = pl.program_id(0)

    @pl.when(i == 0)
    def _():
        col = lax.broadcasted_iota(jnp.int32, (1, _NV), 1)
        acc = jnp.zeros((1, _NV), jnp.float32)
        nc = ivc_ref.shape[0]
        chunk = 100
        for j in range(0, nc, chunk):
            ids = ivc_ref[j:j + chunk, :]
            hit = (ids == col).astype(jnp.float32)
            acc = jnp.maximum(acc, jnp.max(hit, axis=0, keepdims=True))
        cmask_ref[...] = acc

    geo = geo_ref[...]
    vr = vr_ref[...]
    vt = vt_ref[...]

    sqr = jnp.sum(vr * vr, axis=1, keepdims=True)
    sqa = jnp.sum(vt * vt, axis=0, keepdims=True)
    prod = jnp.dot(vr, vt, preferred_element_type=jnp.float32)
    d2 = sqr + sqa - 2.0 * prod
    dist = jnp.sqrt(jnp.maximum(d2, 1e-12))
    distm = jnp.where(geo < _GEO_THRESH, 1e5, dist)

    minv = jnp.min(distm, axis=1, keepdims=True)
    coli = lax.broadcasted_iota(jnp.int32, (_R, _NV), 1)
    idx = jnp.min(jnp.where(distm <= minv, coli, _NV), axis=1, keepdims=True)

    cmask = cmask_ref[...]
    gdi = jnp.min(jnp.where(cmask > 0.0, geo, 1e5), axis=1, keepdims=True)

    minv_ref[...] = minv
    mini_ref[...] = idx
    gdi_ref[...] = gdi


def _stream_minmax(geodist, v, init_verts_in_contact):
    vt = v.T
    vr = jnp.pad(v, ((0, _NVP - _NV), (0, 0)))
    ivc = init_verts_in_contact.reshape(-1, 1).astype(jnp.int32)

    out = pl.pallas_call(
        _stream_body,
        grid=(_NBLK,),
        in_specs=[
            pl.BlockSpec((_R, _NV), lambda i: (i, 0)),
            pl.BlockSpec((_R, 3), lambda i: (i, 0)),
            pl.BlockSpec((3, _NV), lambda i: (0, 0)),
            pl.BlockSpec((ivc.shape[0], 1), lambda i: (0, 0)),
        ],
        out_specs=[
            pl.BlockSpec((_R, 1), lambda i: (i, 0)),
            pl.BlockSpec((_R, 1), lambda i: (i, 0)),
            pl.BlockSpec((_R, 1), lambda i: (i, 0)),
        ],
        out_shape=[
            jax.ShapeDtypeStruct((_NVP, 1), jnp.float32),
            jax.ShapeDtypeStruct((_NVP, 1), jnp.int32),
            jax.ShapeDtypeStruct((_NVP, 1), jnp.float32),
        ],
        scratch_shapes=[pltpu.VMEM((1, _NV), jnp.float32)],
    )(geodist, vr, vt, ivc)
    minv, mini, gdi = out
    return minv.reshape(_NVP), mini.reshape(_NVP), gdi.reshape(_NVP)


_SC_OUT = [
    jax.ShapeDtypeStruct((_NVP,), jnp.float32),
    jax.ShapeDtypeStruct((_NVP,), jnp.float32),
    jax.ShapeDtypeStruct((_NVP,), jnp.float32),
    jax.ShapeDtypeStruct((_NVP,), jnp.float32),
    jax.ShapeDtypeStruct((_NVP,), jnp.float32),
    jax.ShapeDtypeStruct((_NVP,), jnp.float32),
    jax.ShapeDtypeStruct((_DSP,), jnp.float32),
    jax.ShapeDtypeStruct((_DSP,), jnp.float32),
    jax.ShapeDtypeStruct((_DSP,), jnp.float32),
    jax.ShapeDtypeStruct((_HCPAD,), jnp.float32),
    jax.ShapeDtypeStruct((_HCPAD,), jnp.float32),
]

_SC_SCRATCH = [
    pltpu.VMEM((_NVP,), jnp.float32),
    pltpu.VMEM((_NVP,), jnp.float32),
    pltpu.VMEM((_NVP,), jnp.float32),
    pltpu.VMEM((_NVP,), jnp.float32),
    pltpu.VMEM((_NVP,), jnp.float32),
    pltpu.VMEM((_NVP,), jnp.float32),
    pltpu.VMEM((_HNF,), jnp.int32),
    pltpu.VMEM((_HNF,), jnp.int32),
    pltpu.VMEM((_HNF,), jnp.int32),
    pltpu.VMEM((_NVP,), jnp.int32),
    pltpu.VMEM((_NVP,), jnp.float32),
    pltpu.VMEM((_NVP,), jnp.float32),
    pltpu.VMEM((_NVP,), jnp.float32),
    pltpu.VMEM((_NVP,), jnp.float32),
    pltpu.VMEM((_NVP,), jnp.float32),
    pltpu.VMEM((_DSP,), jnp.int32),
    pltpu.VMEM((_HCPAD,), jnp.int32),
    pltpu.VMEM((_DSP,), jnp.float32),
    pltpu.VMEM((_DSP,), jnp.float32),
    pltpu.VMEM((_DSP,), jnp.float32),
    pltpu.VMEM((_HCPAD,), jnp.float32),
    pltpu.VMEM((_HCPAD,), jnp.float32),
]


@functools.partial(
    pl.kernel,
    out_type=_SC_OUT,
    mesh=plsc.VectorSubcoreMesh(core_axis_name="c", subcore_axis_name="s"),
    scratch_types=_SC_SCRATCH,
    compiler_params=pltpu.CompilerParams(needs_layout_passes=False),
)
def _sc_epilogue(vx, vy, vz, f0, f1, f2, mini, minv, gdi, ds, hcp,
                 vnx_o, vny_o, vnz_o, insf_o, ngd_o, gn2_o,
                 mds_o, wds_o, ids_o, mh_o, ih_o,
                 vxv, vyv, vzv, axv, ayv, azv, fb0, fb1, fb2,
                 miniv, minvv, gdiv, insv, ngdv, gn2v,
                 dsv, hcpv, mdsv, wdsv, idsv, mhv, ihv):
    cid = lax.axis_index("c")
    sid = lax.axis_index("s")

    @pl.when(jnp.logical_and(cid == 0, sid == 0))
    def _():
        pltpu.sync_copy(vx, vxv)
        pltpu.sync_copy(vy, vyv)
        pltpu.sync_copy(vz, vzv)
        pltpu.sync_copy(mini, miniv)
        pltpu.sync_copy(minv, minvv)
        pltpu.sync_copy(gdi, gdiv)
        pltpu.sync_copy(ds, dsv)
        pltpu.sync_copy(hcp, hcpv)

        zero16 = jnp.zeros((16,), jnp.float32)

        def zbody(i, c):
            axv[pl.ds(i * 16, 16)] = zero16
            ayv[pl.ds(i * 16, 16)] = zero16
            azv[pl.ds(i * 16, 16)] = zero16
            insv[pl.ds(i * 16, 16)] = zero16
            return c
        lax.fori_loop(0, _NVP // 16, zbody, 0)

        for h in range(_NFP // _HNF):
            pltpu.sync_copy(f0.at[pl.ds(h * _HNF, _HNF)], fb0)
            pltpu.sync_copy(f1.at[pl.ds(h * _HNF, _HNF)], fb1)
            pltpu.sync_copy(f2.at[pl.ds(h * _HNF, _HNF)], fb2)

            def fbody(i, c):
                i0 = fb0[pl.ds(i * 16, 16)]
                i1 = fb1[pl.ds(i * 16, 16)]
                i2 = fb2[pl.ds(i * 16, 16)]
                x0 = plsc.load_gather(vxv, [i0])
                y0 = plsc.load_gather(vyv, [i0])
                z0 = plsc.load_gather(vzv, [i0])
                x1 = plsc.load_gather(vxv, [i1])
                y1 = plsc.load_gather(vyv, [i1])
                z1 = plsc.load_gather(vzv, [i1])
                x2 = plsc.load_gather(vxv, [i2])
                y2 = plsc.load_gather(vyv, [i2])
                z2 = plsc.load_gather(vzv, [i2])
                e1x, e1y, e1z = x1 - x0, y1 - y0, z1 - z0
                e2x, e2y, e2z = x2 - x0, y2 - y0, z2 - z0
                fnx = e1y * e2z - e1z * e2y
                fny = e1z * e2x - e1x * e2z
                fnz = e1x * e2y - e1y * e2x
                for ii in (i0, i1, i2):
                    plsc.addupdate_scatter(axv, [ii], fnx)
                    plsc.addupdate_scatter(ayv, [ii], fny)
                    plsc.addupdate_scatter(azv, [ii], fnz)
                return c
            lax.fori_loop(0, _HNF // 16, fbody, 0)

        def dbody(i, c):
            d = dsv[pl.ds(i * 16, 16)]
            nr = plsc.load_gather(miniv, [d])
            nr = jnp.minimum(jnp.maximum(nr, 0), _NV - 1)
            vxd = plsc.load_gather(vxv, [d])
            vyd = plsc.load_gather(vyv, [d])
            vzd = plsc.load_gather(vzv, [d])
            vxn = plsc.load_gather(vxv, [nr])
            vyn = plsc.load_gather(vyv, [nr])
            vzn = plsc.load_gather(vzv, [nr])
            nx = plsc.load_gather(axv, [nr])
            ny = plsc.load_gather(ayv, [nr])
            nz = plsc.load_gather(azv, [nr])
            ext = (vxd - vxn) * nx + (vyd - vyn) * ny + (vzd - vzn) * nz
            flag = jnp.where(ext > 0.0, 0.0, 1.0)
            plsc.addupdate_scatter(insv, [d], flag)
            return c
        lax.fori_loop(0, _DSP // 16, dbody, 0)

        def d2body(i, c):
            sl = pl.ds(i * 16, 16)
            d = dsv[sl]
            mdsv[sl] = plsc.load_gather(minvv, [d])
            wdsv[sl] = plsc.load_gather(gdiv, [d])
            idsv[sl] = plsc.load_gather(insv, [d])
            return c
        lax.fori_loop(0, _DSP // 16, d2body, 0)

        def hbody(i, c):
            sl = pl.ds(i * 16, 16)
            hh = hcpv[sl]
            mhv[sl] = plsc.load_gather(minvv, [hh])
            ihv[sl] = plsc.load_gather(insv, [hh])
            return c
        lax.fori_loop(0, _HCPAD // 16, hbody, 0)

        def nbody(i, c):
            sl = pl.ds(i * 16, 16)
            aix = axv[sl]
            aiy = ayv[sl]
            aiz = azv[sl]
            ii = miniv[sl]
            ii = jnp.minimum(jnp.maximum(ii, 0), _NV - 1)
            gx = plsc.load_gather(axv, [ii])
            gy = plsc.load_gather(ayv, [ii])
            gz = plsc.load_gather(azv, [ii])
            ngdv[sl] = aix * gx + aiy * gy + aiz * gz
            gn2v[sl] = gx * gx + gy * gy + gz * gz
            return c
        lax.fori_loop(0, _NVP // 16, nbody, 0)

        pltpu.sync_copy(axv, vnx_o)
        pltpu.sync_copy(ayv, vny_o)
        pltpu.sync_copy(azv, vnz_o)
        pltpu.sync_copy(insv, insf_o)
        pltpu.sync_copy(ngdv, ngd_o)
        pltpu.sync_copy(gn2v, gn2_o)
        pltpu.sync_copy(mdsv, mds_o)
        pltpu.sync_copy(wdsv, wds_o)
        pltpu.sync_copy(idsv, ids_o)
        pltpu.sync_copy(mhv, mh_o)
        pltpu.sync_copy(ihv, ih_o)


def _mmean(vals, mask):
    s = jnp.sum(jnp.where(mask, vals, 0.0))
    c = jnp.sum(jnp.where(mask, 1.0, 0.0))
    return jnp.where(c > 0, s / jnp.maximum(c, 1.0), 0.0)


def _assemble_body(minv_ref, gdi_ref, vnx_ref, vny_ref, vnz_ref, insf_ref,
                   ngd_ref, gn2_ref, mds_ref, wds_ref, ids_ref, mh_ref,
                   ih_ref, hw_ref, vt_ref, ivt_ref, bp_ref, ip_ref, lhp_ref,
                   rhp_ref, out_ref):
    nvi = lax.broadcasted_iota(jnp.int32, (1, _NVP), 1)
    valid_v = nvi < _NV

    minv = minv_ref[...]
    gdi = gdi_ref[...]

    dsi = lax.broadcasted_iota(jnp.int32, (1, _DSP), 1)
    valid_ds = dsi < _DS
    inside_ds = ids_ref[...] > 0.0
    w_out = 1.0 / (5.0 * wds_ref[...] + 1.0)
    vals_out = _A1 * w_out * jnp.tanh(mds_ref[...] / _A2)
    contactloss = _CONTACT_W * _mmean(
        vals_out, jnp.logical_and(jnp.logical_not(inside_ds), valid_ds))

    inside_v = insf_ref[...] > 0.0
    vals_in = _B1 * jnp.tanh(minv / _B2)
    insideloss = _INSIDE_W * _mmean(vals_in, jnp.logical_and(inside_v, valid_v))

    hci = lax.broadcasted_iota(jnp.int32, (1, _HCPAD), 1)
    left = hci < _HC // 2
    right = jnp.logical_and(hci >= _HC // 2, hci < _HC)
    hvi = ih_ref[...] > 0.0
    not_hvi = jnp.logical_not(hvi)
    mh = mh_ref[...]
    wv = -0.1 * hw_ref[...] + 1.0
    vals_o = _C1 * jnp.tanh(mh / _C2)
    vals_i = _D1 * jnp.tanh(mh / _D2)
    hclo = (_mmean(wv * vals_o, jnp.logical_and(not_hvi, left)) +
            _mmean(wv * vals_o, jnp.logical_and(not_hvi, right)))
    hcli = (_mmean(vals_i, jnp.logical_and(hvi, left)) +
            _mmean(vals_i, jnp.logical_and(hvi, right)))
    hand_contact_loss = _HCP_W * (hcli + hclo)

    n2 = (vnx_ref[...] * vnx_ref[...] + vny_ref[...] * vny_ref[...] +
          vnz_ref[...] * vnz_ref[...])
    denom = (jnp.sqrt(n2) + 1e-8) * (jnp.sqrt(gn2_ref[...]) + 1e-8)
    ng = ngd_ref[...] / denom
    angle_loss = _ANGLE_W * _mmean(
        1.0 + ng, jnp.logical_and(minv < 0.01, valid_v))

    bp = bp_ref[...]
    ip = ip_ref[...]
    pose_prior_loss = _POSE_W * jnp.sum((bp - ip) * (bp - ip))
    lhp = lhp_ref[...]
    rhp = rhp_ref[...]
    hand_pose_prior_loss = _HAND_POSE_W * (jnp.sum(lhp * lhp) +
                                           jnp.sum(rhp * rhp))

    dvt = ivt_ref[...] - vt_ref[...]
    ov = jnp.sqrt(jnp.sum(dvt * dvt, axis=0, keepdims=True))
    gd = gdi[:, :_NV]
    ow = (2.0 * gd) * (2.0 * gd)
    outsideloss = _OUTSIDE_W * jnp.sum(ov * ow)

    total = (contactloss + insideloss + outsideloss + pose_prior_loss
             + hand_pose_prior_loss + angle_loss + hand_contact_loss)
    out_ref[...] = jnp.reshape(total, (1, 1))


def _assemble(minv, gdi, vnx, vny, vnz, insf, ngd, gn2, mds, wds, ids,
              mh, ih, hw, vt, ivt, bp, ip, lhp, rhp):
    args = [minv.reshape(1, _NVP), gdi.reshape(1, _NVP),
            vnx.reshape(1, _NVP), vny.reshape(1, _NVP), vnz.reshape(1, _NVP),
            insf.reshape(1, _NVP), ngd.reshape(1, _NVP), gn2.reshape(1, _NVP),
            mds.reshape(1, _DSP), wds.reshape(1, _DSP), ids.reshape(1, _DSP),
            mh.reshape(1, _HCPAD), ih.reshape(1, _HCPAD),
            hw.reshape(1, _HCPAD), vt, ivt, bp, ip, lhp, rhp]
    return pl.pallas_call(
        _assemble_body,
        out_shape=jax.ShapeDtypeStruct((1, 1), jnp.float32),
    )(*args)


@jax.jit
def _run(vertices, body_pose, left_hand_pose, right_hand_pose, init_pose,
         init_verts, geodist, hand_contact_prior_weights, ds,
         hand_contact_prior, faces, init_verts_in_contact):
    v = vertices[0]

    minv, mini, gdi = _stream_minmax(geodist, v, init_verts_in_contact)

    vp = jnp.pad(v, ((0, _NVP - _NV), (0, 0)))
    vx, vy, vz = vp[:, 0], vp[:, 1], vp[:, 2]
    fp = jnp.pad(faces.astype(jnp.int32), ((0, _NFP - _NF), (0, 0)))
    f0, f1, f2 = fp[:, 0], fp[:, 1], fp[:, 2]
    dsp = jnp.pad(ds.astype(jnp.int32), (0, _DSP - _DS),
                  constant_values=_NV)
    hcpp = jnp.pad(hand_contact_prior.astype(jnp.int32), (0, _HCPAD - _HC),
                   constant_values=_NV)

    (vnx, vny, vnz, insf, ngd, gn2, mds, wds, ids, mh, ih) = _sc_epilogue(
        vx, vy, vz, f0, f1, f2, mini, minv, gdi, dsp, hcpp)

    hw = jnp.pad(hand_contact_prior_weights, (0, _HCPAD - _HC))
    vt = v.T
    ivt = init_verts[0].T
    out = _assemble(minv, gdi, vnx, vny, vnz, insf, ngd, gn2, mds, wds,
                    ids, mh, ih, hw, vt, ivt, body_pose, init_pose,
                    left_hand_pose, right_hand_pose)
    return out[0, 0]


def kernel(vertices, body_pose, left_hand_pose, right_hand_pose, init_pose,
           init_verts, geodist, hand_contact_prior_weights, ds,
           hand_contact_prior, faces, init_verts_in_contact):
    return _run(vertices, body_pose, left_hand_pose, right_hand_pose,
                init_pose, init_verts, geodist, hand_contact_prior_weights,
                ds, hand_contact_prior, faces, init_verts_in_contact)


# speedup vs baseline: 8.9524x; 1.5585x over previous
import functools
import jax
import jax.numpy as jnp
from jax import lax
from jax.experimental import pallas as pl
from jax.experimental.pallas import tpu as pltpu
from jax.experimental.pallas import tpu_sc as plsc

_NV = 6890
_GEO_THRESH = 0.1
_A1, _A2, _B1, _B2 = 0.04, 0.04, 0.07, 0.06
_C1, _C2, _D1, _D2 = 0.01, 0.01, 0.023, 0.02
_INSIDE_W, _OUTSIDE_W, _CONTACT_W = 0.5, 2.0, 0.5
_HCP_W, _POSE_W, _HAND_POSE_W, _ANGLE_W = 1.0, 0.04, 0.04, 0.01

_R = 256
_NBLK = (_NV + _R - 1) // _R
_NVP = _NBLK * _R

_NF = 13776
_NFP = 14336
_HNF = _NFP // 2
_DS = 1722
_DSP = 1728
_HC = 1556
_HCPAD = 1568


def _stream_body(geo_ref, va_ref, vta_ref, ivc_ref, minv_ref, mini_ref,
                 gdi_ref, amask_ref):
    i = pl.program_id(0)

    @pl.when(i == 0)
    def _():
        col = lax.broadcasted_iota(jnp.int32, (1, _NV), 1)
        acc = jnp.full((1, _NV), 1e10, jnp.float32)
        nc = ivc_ref.shape[0]
        chunk = 100
        for j in range(0, nc, chunk):
            ids = ivc_ref[j:j + chunk, :]
            hit = jnp.max((ids == col).astype(jnp.float32), axis=0,
                          keepdims=True)
            acc = jnp.where(hit > 0.0, 0.0, acc)
        amask_ref[...] = acc

    geo = geo_ref[...]
    va = va_ref[...]
    vta = vta_ref[...]

    d2 = jnp.dot(va, vta, preferred_element_type=jnp.float32)
    d2m = jnp.where(geo < _GEO_THRESH, 1e10, jnp.maximum(d2, 1e-12))

    minv2 = jnp.min(d2m, axis=1, keepdims=True)
    coli = lax.broadcasted_iota(jnp.int32, (_R, _NV), 1)
    idx = jnp.min(jnp.where(d2m <= minv2, coli, _NV), axis=1, keepdims=True)
    minv = jnp.where(minv2 >= 1e10, 1e5, jnp.sqrt(minv2))

    gdi = jnp.min(geo + amask_ref[...], axis=1, keepdims=True)

    minv_ref[...] = minv
    mini_ref[...] = idx
    gdi_ref[...] = gdi


def _stream_minmax(geodist, v, init_verts_in_contact):
    sq = jnp.sum(v * v, axis=1)
    one = jnp.ones((_NV,), jnp.float32)
    zero = jnp.zeros((_NV,), jnp.float32)
    va = jnp.stack([v[:, 0], v[:, 1], v[:, 2], one, sq, zero, zero, zero],
                   axis=1)
    va = jnp.pad(va, ((0, _NVP - _NV), (0, 0)))
    vta = jnp.stack([-2.0 * v[:, 0], -2.0 * v[:, 1], -2.0 * v[:, 2], sq,
                     one, zero, zero, zero], axis=0)
    ivc = init_verts_in_contact.reshape(-1, 1).astype(jnp.int32)

    out = pl.pallas_call(
        _stream_body,
        grid=(_NBLK,),
        in_specs=[
            pl.BlockSpec((_R, _NV), lambda i: (i, 0)),
            pl.BlockSpec((_R, 8), lambda i: (i, 0)),
            pl.BlockSpec((8, _NV), lambda i: (0, 0)),
            pl.BlockSpec((ivc.shape[0], 1), lambda i: (0, 0)),
        ],
        out_specs=[
            pl.BlockSpec((_R, 1), lambda i: (i, 0)),
            pl.BlockSpec((_R, 1), lambda i: (i, 0)),
            pl.BlockSpec((_R, 1), lambda i: (i, 0)),
        ],
        out_shape=[
            jax.ShapeDtypeStruct((_NVP, 1), jnp.float32),
            jax.ShapeDtypeStruct((_NVP, 1), jnp.int32),
            jax.ShapeDtypeStruct((_NVP, 1), jnp.float32),
        ],
        scratch_shapes=[pltpu.VMEM((1, _NV), jnp.float32)],
    )(geodist, va, vta, ivc)
    minv, mini, gdi = out
    return minv.reshape(_NVP), mini.reshape(_NVP), gdi.reshape(_NVP)


_SC_MESH = plsc.VectorSubcoreMesh(core_axis_name="c", subcore_axis_name="s")
_SC_PARAMS = pltpu.CompilerParams(needs_layout_passes=False)


@functools.partial(
    pl.kernel,
    out_type=[
        jax.ShapeDtypeStruct((_NVP,), jnp.float32),
        jax.ShapeDtypeStruct((_NVP,), jnp.float32),
        jax.ShapeDtypeStruct((_NVP,), jnp.float32),
    ],
    mesh=_SC_MESH,
    scratch_types=[
        pltpu.VMEM((_NVP,), jnp.float32),
        pltpu.VMEM((_NVP,), jnp.float32),
        pltpu.VMEM((_NVP,), jnp.float32),
        pltpu.VMEM((_NVP,), jnp.float32),
        pltpu.VMEM((_NVP,), jnp.float32),
        pltpu.VMEM((_NVP,), jnp.float32),
        pltpu.VMEM((_NFP,), jnp.int32),
        pltpu.VMEM((_NFP,), jnp.int32),
        pltpu.VMEM((_NFP,), jnp.int32),
    ],
    compiler_params=_SC_PARAMS,
)
def _sc_normals(vx, vy, vz, f0, f1, f2,
                vnx_o, vny_o, vnz_o,
                vxv, vyv, vzv, axv, ayv, azv, fb0, fb1, fb2):
    cid = lax.axis_index("c")
    sid = lax.axis_index("s")

    @pl.when(jnp.logical_and(cid == 0, sid == 0))
    def _():
        pltpu.sync_copy(vx, vxv)
        pltpu.sync_copy(vy, vyv)
        pltpu.sync_copy(vz, vzv)
        pltpu.sync_copy(f0, fb0)
        pltpu.sync_copy(f1, fb1)
        pltpu.sync_copy(f2, fb2)

        zero16 = jnp.zeros((16,), jnp.float32)

        def zbody(i, c):
            axv[pl.ds(i * 16, 16)] = zero16
            ayv[pl.ds(i * 16, 16)] = zero16
            azv[pl.ds(i * 16, 16)] = zero16
            return c
        lax.fori_loop(0, _NVP // 16, zbody, 0)

        def fbody(i, c):
            i0 = fb0[pl.ds(i * 16, 16)]
            i1 = fb1[pl.ds(i * 16, 16)]
            i2 = fb2[pl.ds(i * 16, 16)]
            x0 = plsc.load_gather(vxv, [i0])
            y0 = plsc.load_gather(vyv, [i0])
            z0 = plsc.load_gather(vzv, [i0])
            x1 = plsc.load_gather(vxv, [i1])
            y1 = plsc.load_gather(vyv, [i1])
            z1 = plsc.load_gather(vzv, [i1])
            x2 = plsc.load_gather(vxv, [i2])
            y2 = plsc.load_gather(vyv, [i2])
            z2 = plsc.load_gather(vzv, [i2])
            e1x, e1y, e1z = x1 - x0, y1 - y0, z1 - z0
            e2x, e2y, e2z = x2 - x0, y2 - y0, z2 - z0
            fnx = e1y * e2z - e1z * e2y
            fny = e1z * e2x - e1x * e2z
            fnz = e1x * e2y - e1y * e2x
            for ii in (i0, i1, i2):
                plsc.addupdate_scatter(axv, [ii], fnx)
                plsc.addupdate_scatter(ayv, [ii], fny)
                plsc.addupdate_scatter(azv, [ii], fnz)
            return c
        lax.fori_loop(0, _NFP // 16, fbody, 0)

        pltpu.sync_copy(axv, vnx_o)
        pltpu.sync_copy(ayv, vny_o)
        pltpu.sync_copy(azv, vnz_o)


@functools.partial(
    pl.kernel,
    out_type=[
        jax.ShapeDtypeStruct((_NVP,), jnp.float32),
        jax.ShapeDtypeStruct((_NVP,), jnp.float32),
        jax.ShapeDtypeStruct((_NVP,), jnp.float32),
        jax.ShapeDtypeStruct((_DSP,), jnp.float32),
        jax.ShapeDtypeStruct((_DSP,), jnp.float32),
        jax.ShapeDtypeStruct((_DSP,), jnp.float32),
        jax.ShapeDtypeStruct((_HCPAD,), jnp.float32),
        jax.ShapeDtypeStruct((_HCPAD,), jnp.float32),
    ],
    mesh=_SC_MESH,
    scratch_types=[
        pltpu.VMEM((_NVP,), jnp.float32),
        pltpu.VMEM((_NVP,), jnp.float32),
        pltpu.VMEM((_NVP,), jnp.float32),
        pltpu.VMEM((_NVP,), jnp.float32),
        pltpu.VMEM((_NVP,), jnp.float32),
        pltpu.VMEM((_NVP,), jnp.float32),
        pltpu.VMEM((_NVP,), jnp.int32),
        pltpu.VMEM((_NVP,), jnp.float32),
        pltpu.VMEM((_NVP,), jnp.float32),
        pltpu.VMEM((_NVP,), jnp.float32),
        pltpu.VMEM((_NVP,), jnp.float32),
        pltpu.VMEM((_NVP,), jnp.float32),
        pltpu.VMEM((_DSP,), jnp.int32),
        pltpu.VMEM((_HCPAD,), jnp.int32),
        pltpu.VMEM((_DSP,), jnp.float32),
        pltpu.VMEM((_DSP,), jnp.float32),
        pltpu.VMEM((_DSP,), jnp.float32),
        pltpu.VMEM((_HCPAD,), jnp.float32),
        pltpu.VMEM((_HCPAD,), jnp.float32),
    ],
    compiler_params=_SC_PARAMS,
)
def _sc_gathers(vx, vy, vz, vnx, vny, vnz, mini, minv, gdi, ds, hcp,
                insf_o, ngd_o, gn2_o, mds_o, wds_o, ids_o, mh_o, ih_o,
                vxv, vyv, vzv, axv, ayv, azv,
                miniv, minvv, gdiv, insv, ngdv, gn2v,
                dsv, hcpv, mdsv, wdsv, idsv, mhv, ihv):
    cid = lax.axis_index("c")
    sid = lax.axis_index("s")

    @pl.when(jnp.logical_and(cid == 0, sid == 0))
    def _():
        pltpu.sync_copy(vx, vxv)
        pltpu.sync_copy(vy, vyv)
        pltpu.sync_copy(vz, vzv)
        pltpu.sync_copy(vnx, axv)
        pltpu.sync_copy(vny, ayv)
        pltpu.sync_copy(vnz, azv)
        pltpu.sync_copy(mini, miniv)
        pltpu.sync_copy(minv, minvv)
        pltpu.sync_copy(gdi, gdiv)
        pltpu.sync_copy(ds, dsv)
        pltpu.sync_copy(hcp, hcpv)

        zero16 = jnp.zeros((16,), jnp.float32)

        def zbody(i, c):
            insv[pl.ds(i * 16, 16)] = zero16
            return c
        lax.fori_loop(0, _NVP // 16, zbody, 0)

        def dbody(i, c):
            d = dsv[pl.ds(i * 16, 16)]
            nr = plsc.load_gather(miniv, [d])
            nr = jnp.minimum(jnp.maximum(nr, 0), _NV - 1)
            vxd = plsc.load_gather(vxv, [d])
            vyd = plsc.load_gather(vyv, [d])
            vzd = plsc.load_gather(vzv, [d])
            vxn = plsc.load_gather(vxv, [nr])
            vyn = plsc.load_gather(vyv, [nr])
            vzn = plsc.load_gather(vzv, [nr])
            nx = plsc.load_gather(axv, [nr])
            ny = plsc.load_gather(ayv, [nr])
            nz = plsc.load_gather(azv, [nr])
            ext = (vxd - vxn) * nx + (vyd - vyn) * ny + (vzd - vzn) * nz
            flag = jnp.where(ext > 0.0, 0.0, 1.0)
            plsc.addupdate_scatter(insv, [d], flag)
            return c
        lax.fori_loop(0, _DSP // 16, dbody, 0)

        def d2body(i, c):
            sl = pl.ds(i * 16, 16)
            d = dsv[sl]
            mdsv[sl] = plsc.load_gather(minvv, [d])
            wdsv[sl] = plsc.load_gather(gdiv, [d])
            idsv[sl] = plsc.load_gather(insv, [d])
            return c
        lax.fori_loop(0, _DSP // 16, d2body, 0)

        def hbody(i, c):
            sl = pl.ds(i * 16, 16)
            hh = hcpv[sl]
            mhv[sl] = plsc.load_gather(minvv, [hh])
            ihv[sl] = plsc.load_gather(insv, [hh])
            return c
        lax.fori_loop(0, _HCPAD // 16, hbody, 0)

        def nbody(i, c):
            sl = pl.ds(i * 16, 16)
            aix = axv[sl]
            aiy = ayv[sl]
            aiz = azv[sl]
            ii = miniv[sl]
            ii = jnp.minimum(jnp.maximum(ii, 0), _NV - 1)
            gx = plsc.load_gather(axv, [ii])
            gy = plsc.load_gather(ayv, [ii])
            gz = plsc.load_gather(azv, [ii])
            ngdv[sl] = aix * gx + aiy * gy + aiz * gz
            gn2v[sl] = gx * gx + gy * gy + gz * gz
            return c
        lax.fori_loop(0, _NVP // 16, nbody, 0)

        pltpu.sync_copy(insv, insf_o)
        pltpu.sync_copy(ngdv, ngd_o)
        pltpu.sync_copy(gn2v, gn2_o)
        pltpu.sync_copy(mdsv, mds_o)
        pltpu.sync_copy(wdsv, wds_o)
        pltpu.sync_copy(idsv, ids_o)
        pltpu.sync_copy(mhv, mh_o)
        pltpu.sync_copy(ihv, ih_o)


def _mmean(vals, mask):
    s = jnp.sum(jnp.where(mask, vals, 0.0))
    c = jnp.sum(jnp.where(mask, 1.0, 0.0))
    return jnp.where(c > 0, s / jnp.maximum(c, 1.0), 0.0)


def _assemble_body(minv_ref, gdi_ref, vnx_ref, vny_ref, vnz_ref, insf_ref,
                   ngd_ref, gn2_ref, mds_ref, wds_ref, ids_ref, mh_ref,
                   ih_ref, hw_ref, vt_ref, ivt_ref, bp_ref, ip_ref, lhp_ref,
                   rhp_ref, out_ref):
    nvi = lax.broadcasted_iota(jnp.int32, (1, _NVP), 1)
    valid_v = nvi < _NV

    minv = minv_ref[...]
    gdi = gdi_ref[...]

    dsi = lax.broadcasted_iota(jnp.int32, (1, _DSP), 1)
    valid_ds = dsi < _DS
    inside_ds = ids_ref[...] > 0.0
    w_out = 1.0 / (5.0 * wds_ref[...] + 1.0)
    vals_out = _A1 * w_out * jnp.tanh(mds_ref[...] / _A2)
    contactloss = _CONTACT_W * _mmean(
        vals_out, jnp.logical_and(jnp.logical_not(inside_ds), valid_ds))

    inside_v = insf_ref[...] > 0.0
    vals_in = _B1 * jnp.tanh(minv / _B2)
    insideloss = _INSIDE_W * _mmean(vals_in, jnp.logical_and(inside_v, valid_v))

    hci = lax.broadcasted_iota(jnp.int32, (1, _HCPAD), 1)
    left = hci < _HC // 2
    right = jnp.logical_and(hci >= _HC // 2, hci < _HC)
    hvi = ih_ref[...] > 0.0
    not_hvi = jnp.logical_not(hvi)
    mh = mh_ref[...]
    wv = -0.1 * hw_ref[...] + 1.0
    vals_o = _C1 * jnp.tanh(mh / _C2)
    vals_i = _D1 * jnp.tanh(mh / _D2)
    hclo = (_mmean(wv * vals_o, jnp.logical_and(not_hvi, left)) +
            _mmean(wv * vals_o, jnp.logical_and(not_hvi, right)))
    hcli = (_mmean(vals_i, jnp.logical_and(hvi, left)) +
            _mmean(vals_i, jnp.logical_and(hvi, right)))
    hand_contact_loss = _HCP_W * (hcli + hclo)

    n2 = (vnx_ref[...] * vnx_ref[...] + vny_ref[...] * vny_ref[...] +
          vnz_ref[...] * vnz_ref[...])
    denom = (jnp.sqrt(n2) + 1e-8) * (jnp.sqrt(gn2_ref[...]) + 1e-8)
    ng = ngd_ref[...] / denom
    angle_loss = _ANGLE_W * _mmean(
        1.0 + ng, jnp.logical_and(minv < 0.01, valid_v))

    bp = bp_ref[...]
    ip = ip_ref[...]
    pose_prior_loss = _POSE_W * jnp.sum((bp - ip) * (bp - ip))
    lhp = lhp_ref[...]
    rhp = rhp_ref[...]
    hand_pose_prior_loss = _HAND_POSE_W * (jnp.sum(lhp * lhp) +
                                           jnp.sum(rhp * rhp))

    dvt = ivt_ref[...] - vt_ref[...]
    ov = jnp.sqrt(jnp.sum(dvt * dvt, axis=0, keepdims=True))
    gd = gdi[:, :_NV]
    ow = (2.0 * gd) * (2.0 * gd)
    outsideloss = _OUTSIDE_W * jnp.sum(ov * ow)

    total = (contactloss + insideloss + outsideloss + pose_prior_loss
             + hand_pose_prior_loss + angle_loss + hand_contact_loss)
    out_ref[...] = jnp.reshape(total, (1, 1))


def _assemble(minv, gdi, vnx, vny, vnz, insf, ngd, gn2, mds, wds, ids,
              mh, ih, hw, vt, ivt, bp, ip, lhp, rhp):
    args = [minv.reshape(1, _NVP), gdi.reshape(1, _NVP),
            vnx.reshape(1, _NVP), vny.reshape(1, _NVP), vnz.reshape(1, _NVP),
            insf.reshape(1, _NVP), ngd.reshape(1, _NVP), gn2.reshape(1, _NVP),
            mds.reshape(1, _DSP), wds.reshape(1, _DSP), ids.reshape(1, _DSP),
            mh.reshape(1, _HCPAD), ih.reshape(1, _HCPAD),
            hw.reshape(1, _HCPAD), vt, ivt, bp, ip, lhp, rhp]
    return pl.pallas_call(
        _assemble_body,
        out_shape=jax.ShapeDtypeStruct((1, 1), jnp.float32),
    )(*args)


@jax.jit
def _run(vertices, body_pose, left_hand_pose, right_hand_pose, init_pose,
         init_verts, geodist, hand_contact_prior_weights, ds,
         hand_contact_prior, faces, init_verts_in_contact):
    v = vertices[0]

    minv, mini, gdi = _stream_minmax(geodist, v, init_verts_in_contact)

    vp = jnp.pad(v, ((0, _NVP - _NV), (0, 0)))
    vx, vy, vz = vp[:, 0], vp[:, 1], vp[:, 2]
    fp = jnp.pad(faces.astype(jnp.int32), ((0, _NFP - _NF), (0, 0)))
    f0, f1, f2 = fp[:, 0], fp[:, 1], fp[:, 2]
    dsp = jnp.pad(ds.astype(jnp.int32), (0, _DSP - _DS),
                  constant_values=_NV)
    hcpp = jnp.pad(hand_contact_prior.astype(jnp.int32), (0, _HCPAD - _HC),
                   constant_values=_NV)

    vnx, vny, vnz = _sc_normals(vx, vy, vz, f0, f1, f2)
    (insf, ngd, gn2, mds, wds, ids, mh, ih) = _sc_gathers(
        vx, vy, vz, vnx, vny, vnz, mini, minv, gdi, dsp, hcpp)

    hw = jnp.pad(hand_contact_prior_weights, (0, _HCPAD - _HC))
    vt = v.T
    ivt = init_verts[0].T
    out = _assemble(minv, gdi, vnx, vny, vnz, insf, ngd, gn2, mds, wds,
                    ids, mh, ih, hw, vt, ivt, body_pose, init_pose,
                    left_hand_pose, right_hand_pose)
    return out[0, 0]


def kernel(vertices, body_pose, left_hand_pose, right_hand_pose, init_pose,
           init_verts, geodist, hand_contact_prior_weights, ds,
           hand_contact_prior, faces, init_verts_in_contact):
    return _run(vertices, body_pose, left_hand_pose, right_hand_pose,
                init_pose, init_verts, geodist, hand_contact_prior_weights,
                ds, hand_contact_prior, faces, init_verts_in_contact)


# speedup vs baseline: 9.4008x; 1.0501x over previous
import functools
import jax
import jax.numpy as jnp
from jax import lax
from jax.experimental import pallas as pl
from jax.experimental.pallas import tpu as pltpu
from jax.experimental.pallas import tpu_sc as plsc

_NV = 6890
_GEO_THRESH = 0.1
_A1, _A2, _B1, _B2 = 0.04, 0.04, 0.07, 0.06
_C1, _C2, _D1, _D2 = 0.01, 0.01, 0.023, 0.02
_INSIDE_W, _OUTSIDE_W, _CONTACT_W = 0.5, 2.0, 0.5
_HCP_W, _POSE_W, _HAND_POSE_W, _ANGLE_W = 1.0, 0.04, 0.04, 0.01

_R = 896
_NBLK = (_NV + _R - 1) // _R
_NVP = _NBLK * _R

_NF = 13776
_NFP = 14336
_HNF = _NFP // 2
_DS = 1722
_DSP = 1728
_HC = 1556
_HCPAD = 1568


def _stream_body(geo_ref, va_ref, vta_ref, ivc_ref, minv_ref, mini_ref,
                 gdi_ref, amask_ref):
    i = pl.program_id(0)

    @pl.when(i == 0)
    def _():
        col = lax.broadcasted_iota(jnp.int32, (1, _NV), 1)
        acc = jnp.full((1, _NV), 1e10, jnp.float32)
        nc = ivc_ref.shape[0]
        chunk = 100
        for j in range(0, nc, chunk):
            ids = ivc_ref[j:j + chunk, :]
            hit = jnp.max((ids == col).astype(jnp.float32), axis=0,
                          keepdims=True)
            acc = jnp.where(hit > 0.0, 0.0, acc)
        amask_ref[...] = acc

    geo = geo_ref[...]
    va = va_ref[...]
    vta = vta_ref[...]

    d2 = jnp.dot(va, vta, preferred_element_type=jnp.float32)
    d2m = jnp.where(geo < _GEO_THRESH, 1e10, jnp.maximum(d2, 1e-12))

    minv2 = jnp.min(d2m, axis=1, keepdims=True)
    idx = jnp.argmin(d2m, axis=1).astype(jnp.int32)[:, None]
    minv = jnp.where(minv2 >= 1e10, 1e5, jnp.sqrt(minv2))

    gdi = jnp.min(geo + amask_ref[...], axis=1, keepdims=True)

    minv_ref[...] = minv
    mini_ref[...] = idx
    gdi_ref[...] = gdi


def _stream_minmax(geodist, v, init_verts_in_contact):
    sq = jnp.sum(v * v, axis=1)
    one = jnp.ones((_NV,), jnp.float32)
    zero = jnp.zeros((_NV,), jnp.float32)
    va = jnp.stack([v[:, 0], v[:, 1], v[:, 2], one, sq, zero, zero, zero],
                   axis=1)
    va = jnp.pad(va, ((0, _NVP - _NV), (0, 0)))
    vta = jnp.stack([-2.0 * v[:, 0], -2.0 * v[:, 1], -2.0 * v[:, 2], sq,
                     one, zero, zero, zero], axis=0)
    ivc = init_verts_in_contact.reshape(-1, 1).astype(jnp.int32)

    out = pl.pallas_call(
        _stream_body,
        grid=(_NBLK,),
        in_specs=[
            pl.BlockSpec((_R, _NV), lambda i: (i, 0)),
            pl.BlockSpec((_R, 8), lambda i: (i, 0)),
            pl.BlockSpec((8, _NV), lambda i: (0, 0)),
            pl.BlockSpec((ivc.shape[0], 1), lambda i: (0, 0)),
        ],
        out_specs=[
            pl.BlockSpec((_R, 1), lambda i: (i, 0)),
            pl.BlockSpec((_R, 1), lambda i: (i, 0)),
            pl.BlockSpec((_R, 1), lambda i: (i, 0)),
        ],
        out_shape=[
            jax.ShapeDtypeStruct((_NVP, 1), jnp.float32),
            jax.ShapeDtypeStruct((_NVP, 1), jnp.int32),
            jax.ShapeDtypeStruct((_NVP, 1), jnp.float32),
        ],
        scratch_shapes=[pltpu.VMEM((1, _NV), jnp.float32)],
    )(geodist, va, vta, ivc)
    minv, mini, gdi = out
    return minv.reshape(_NVP), mini.reshape(_NVP), gdi.reshape(_NVP)


_SC_MESH = plsc.VectorSubcoreMesh(core_axis_name="c", subcore_axis_name="s")
_SC_PARAMS = pltpu.CompilerParams(needs_layout_passes=False)


@functools.partial(
    pl.kernel,
    out_type=[
        jax.ShapeDtypeStruct((_NVP,), jnp.float32),
        jax.ShapeDtypeStruct((_NVP,), jnp.float32),
        jax.ShapeDtypeStruct((_NVP,), jnp.float32),
    ],
    mesh=_SC_MESH,
    scratch_types=[
        pltpu.VMEM((_NVP,), jnp.float32),
        pltpu.VMEM((_NVP,), jnp.float32),
        pltpu.VMEM((_NVP,), jnp.float32),
        pltpu.VMEM((_NVP,), jnp.float32),
        pltpu.VMEM((_NVP,), jnp.float32),
        pltpu.VMEM((_NVP,), jnp.float32),
        pltpu.VMEM((_NFP,), jnp.int32),
        pltpu.VMEM((_NFP,), jnp.int32),
        pltpu.VMEM((_NFP,), jnp.int32),
    ],
    compiler_params=_SC_PARAMS,
)
def _sc_normals(vx, vy, vz, f0, f1, f2,
                vnx_o, vny_o, vnz_o,
                vxv, vyv, vzv, axv, ayv, azv, fb0, fb1, fb2):
    cid = lax.axis_index("c")
    sid = lax.axis_index("s")

    @pl.when(jnp.logical_and(cid == 0, sid == 0))
    def _():
        pltpu.sync_copy(vx, vxv)
        pltpu.sync_copy(vy, vyv)
        pltpu.sync_copy(vz, vzv)
        pltpu.sync_copy(f0, fb0)
        pltpu.sync_copy(f1, fb1)
        pltpu.sync_copy(f2, fb2)

        zero16 = jnp.zeros((16,), jnp.float32)

        def zbody(i, c):
            axv[pl.ds(i * 16, 16)] = zero16
            ayv[pl.ds(i * 16, 16)] = zero16
            azv[pl.ds(i * 16, 16)] = zero16
            return c
        lax.fori_loop(0, _NVP // 16, zbody, 0)

        def fbody(i, c):
            i0 = fb0[pl.ds(i * 16, 16)]
            i1 = fb1[pl.ds(i * 16, 16)]
            i2 = fb2[pl.ds(i * 16, 16)]
            x0 = plsc.load_gather(vxv, [i0])
            y0 = plsc.load_gather(vyv, [i0])
            z0 = plsc.load_gather(vzv, [i0])
            x1 = plsc.load_gather(vxv, [i1])
            y1 = plsc.load_gather(vyv, [i1])
            z1 = plsc.load_gather(vzv, [i1])
            x2 = plsc.load_gather(vxv, [i2])
            y2 = plsc.load_gather(vyv, [i2])
            z2 = plsc.load_gather(vzv, [i2])
            e1x, e1y, e1z = x1 - x0, y1 - y0, z1 - z0
            e2x, e2y, e2z = x2 - x0, y2 - y0, z2 - z0
            fnx = e1y * e2z - e1z * e2y
            fny = e1z * e2x - e1x * e2z
            fnz = e1x * e2y - e1y * e2x
            for ii in (i0, i1, i2):
                plsc.addupdate_scatter(axv, [ii], fnx)
                plsc.addupdate_scatter(ayv, [ii], fny)
                plsc.addupdate_scatter(azv, [ii], fnz)
            return c
        lax.fori_loop(0, _NFP // 16, fbody, 0)

        pltpu.sync_copy(axv, vnx_o)
        pltpu.sync_copy(ayv, vny_o)
        pltpu.sync_copy(azv, vnz_o)


@functools.partial(
    pl.kernel,
    out_type=[
        jax.ShapeDtypeStruct((_NVP,), jnp.float32),
        jax.ShapeDtypeStruct((_NVP,), jnp.float32),
        jax.ShapeDtypeStruct((_NVP,), jnp.float32),
        jax.ShapeDtypeStruct((_DSP,), jnp.float32),
        jax.ShapeDtypeStruct((_DSP,), jnp.float32),
        jax.ShapeDtypeStruct((_DSP,), jnp.float32),
        jax.ShapeDtypeStruct((_HCPAD,), jnp.float32),
        jax.ShapeDtypeStruct((_HCPAD,), jnp.float32),
    ],
    mesh=_SC_MESH,
    scratch_types=[
        pltpu.VMEM((_NVP,), jnp.float32),
        pltpu.VMEM((_NVP,), jnp.float32),
        pltpu.VMEM((_NVP,), jnp.float32),
        pltpu.VMEM((_NVP,), jnp.float32),
        pltpu.VMEM((_NVP,), jnp.float32),
        pltpu.VMEM((_NVP,), jnp.float32),
        pltpu.VMEM((_NVP,), jnp.int32),
        pltpu.VMEM((_NVP,), jnp.float32),
        pltpu.VMEM((_NVP,), jnp.float32),
        pltpu.VMEM((_NVP,), jnp.float32),
        pltpu.VMEM((_NVP,), jnp.float32),
        pltpu.VMEM((_NVP,), jnp.float32),
        pltpu.VMEM((_DSP,), jnp.int32),
        pltpu.VMEM((_HCPAD,), jnp.int32),
        pltpu.VMEM((_DSP,), jnp.float32),
        pltpu.VMEM((_DSP,), jnp.float32),
        pltpu.VMEM((_DSP,), jnp.float32),
        pltpu.VMEM((_HCPAD,), jnp.float32),
        pltpu.VMEM((_HCPAD,), jnp.float32),
    ],
    compiler_params=_SC_PARAMS,
)
def _sc_gathers(vx, vy, vz, vnx, vny, vnz, mini, minv, gdi, ds, hcp,
                insf_o, ngd_o, gn2_o, mds_o, wds_o, ids_o, mh_o, ih_o,
                vxv, vyv, vzv, axv, ayv, azv,
                miniv, minvv, gdiv, insv, ngdv, gn2v,
                dsv, hcpv, mdsv, wdsv, idsv, mhv, ihv):
    cid = lax.axis_index("c")
    sid = lax.axis_index("s")

    @pl.when(jnp.logical_and(cid == 0, sid == 0))
    def _():
        pltpu.sync_copy(vx, vxv)
        pltpu.sync_copy(vy, vyv)
        pltpu.sync_copy(vz, vzv)
        pltpu.sync_copy(vnx, axv)
        pltpu.sync_copy(vny, ayv)
        pltpu.sync_copy(vnz, azv)
        pltpu.sync_copy(mini, miniv)
        pltpu.sync_copy(minv, minvv)
        pltpu.sync_copy(gdi, gdiv)
        pltpu.sync_copy(ds, dsv)
        pltpu.sync_copy(hcp, hcpv)

        zero16 = jnp.zeros((16,), jnp.float32)

        def zbody(i, c):
            insv[pl.ds(i * 16, 16)] = zero16
            return c
        lax.fori_loop(0, _NVP // 16, zbody, 0)

        def dbody(i, c):
            d = dsv[pl.ds(i * 16, 16)]
            nr = plsc.load_gather(miniv, [d])
            nr = jnp.minimum(jnp.maximum(nr, 0), _NV - 1)
            vxd = plsc.load_gather(vxv, [d])
            vyd = plsc.load_gather(vyv, [d])
            vzd = plsc.load_gather(vzv, [d])
            vxn = plsc.load_gather(vxv, [nr])
            vyn = plsc.load_gather(vyv, [nr])
            vzn = plsc.load_gather(vzv, [nr])
            nx = plsc.load_gather(axv, [nr])
            ny = plsc.load_gather(ayv, [nr])
            nz = plsc.load_gather(azv, [nr])
            ext = (vxd - vxn) * nx + (vyd - vyn) * ny + (vzd - vzn) * nz
            flag = jnp.where(ext > 0.0, 0.0, 1.0)
            plsc.addupdate_scatter(insv, [d], flag)
            return c
        lax.fori_loop(0, _DSP // 16, dbody, 0)

        def d2body(i, c):
            sl = pl.ds(i * 16, 16)
            d = dsv[sl]
            mdsv[sl] = plsc.load_gather(minvv, [d])
            wdsv[sl] = plsc.load_gather(gdiv, [d])
            idsv[sl] = plsc.load_gather(insv, [d])
            return c
        lax.fori_loop(0, _DSP // 16, d2body, 0)

        def hbody(i, c):
            sl = pl.ds(i * 16, 16)
            hh = hcpv[sl]
            mhv[sl] = plsc.load_gather(minvv, [hh])
            ihv[sl] = plsc.load_gather(insv, [hh])
            return c
        lax.fori_loop(0, _HCPAD // 16, hbody, 0)

        def nbody(i, c):
            sl = pl.ds(i * 16, 16)
            aix = axv[sl]
            aiy = ayv[sl]
            aiz = azv[sl]
            ii = miniv[sl]
            ii = jnp.minimum(jnp.maximum(ii, 0), _NV - 1)
            gx = plsc.load_gather(axv, [ii])
            gy = plsc.load_gather(ayv, [ii])
            gz = plsc.load_gather(azv, [ii])
            ngdv[sl] = aix * gx + aiy * gy + aiz * gz
            gn2v[sl] = gx * gx + gy * gy + gz * gz
            return c
        lax.fori_loop(0, _NVP // 16, nbody, 0)

        pltpu.sync_copy(insv, insf_o)
        pltpu.sync_copy(ngdv, ngd_o)
        pltpu.sync_copy(gn2v, gn2_o)
        pltpu.sync_copy(mdsv, mds_o)
        pltpu.sync_copy(wdsv, wds_o)
        pltpu.sync_copy(idsv, ids_o)
        pltpu.sync_copy(mhv, mh_o)
        pltpu.sync_copy(ihv, ih_o)


def _mmean(vals, mask):
    s = jnp.sum(jnp.where(mask, vals, 0.0))
    c = jnp.sum(jnp.where(mask, 1.0, 0.0))
    return jnp.where(c > 0, s / jnp.maximum(c, 1.0), 0.0)


def _assemble_body(minv_ref, gdi_ref, vnx_ref, vny_ref, vnz_ref, insf_ref,
                   ngd_ref, gn2_ref, mds_ref, wds_ref, ids_ref, mh_ref,
                   ih_ref, hw_ref, vt_ref, ivt_ref, bp_ref, ip_ref, lhp_ref,
                   rhp_ref, out_ref):
    nvi = lax.broadcasted_iota(jnp.int32, (1, _NVP), 1)
    valid_v = nvi < _NV

    minv = minv_ref[...]
    gdi = gdi_ref[...]

    dsi = lax.broadcasted_iota(jnp.int32, (1, _DSP), 1)
    valid_ds = dsi < _DS
    inside_ds = ids_ref[...] > 0.0
    w_out = 1.0 / (5.0 * wds_ref[...] + 1.0)
    vals_out = _A1 * w_out * jnp.tanh(mds_ref[...] / _A2)
    contactloss = _CONTACT_W * _mmean(
        vals_out, jnp.logical_and(jnp.logical_not(inside_ds), valid_ds))

    inside_v = insf_ref[...] > 0.0
    vals_in = _B1 * jnp.tanh(minv / _B2)
    insideloss = _INSIDE_W * _mmean(vals_in, jnp.logical_and(inside_v, valid_v))

    hci = lax.broadcasted_iota(jnp.int32, (1, _HCPAD), 1)
    left = hci < _HC // 2
    right = jnp.logical_and(hci >= _HC // 2, hci < _HC)
    hvi = ih_ref[...] > 0.0
    not_hvi = jnp.logical_not(hvi)
    mh = mh_ref[...]
    wv = -0.1 * hw_ref[...] + 1.0
    vals_o = _C1 * jnp.tanh(mh / _C2)
    vals_i = _D1 * jnp.tanh(mh / _D2)
    hclo = (_mmean(wv * vals_o, jnp.logical_and(not_hvi, left)) +
            _mmean(wv * vals_o, jnp.logical_and(not_hvi, right)))
    hcli = (_mmean(vals_i, jnp.logical_and(hvi, left)) +
            _mmean(vals_i, jnp.logical_and(hvi, right)))
    hand_contact_loss = _HCP_W * (hcli + hclo)

    n2 = (vnx_ref[...] * vnx_ref[...] + vny_ref[...] * vny_ref[...] +
          vnz_ref[...] * vnz_ref[...])
    denom = (jnp.sqrt(n2) + 1e-8) * (jnp.sqrt(gn2_ref[...]) + 1e-8)
    ng = ngd_ref[...] / denom
    angle_loss = _ANGLE_W * _mmean(
        1.0 + ng, jnp.logical_and(minv < 0.01, valid_v))

    bp = bp_ref[...]
    ip = ip_ref[...]
    pose_prior_loss = _POSE_W * jnp.sum((bp - ip) * (bp - ip))
    lhp = lhp_ref[...]
    rhp = rhp_ref[...]
    hand_pose_prior_loss = _HAND_POSE_W * (jnp.sum(lhp * lhp) +
                                           jnp.sum(rhp * rhp))

    dvt = ivt_ref[...] - vt_ref[...]
    ov = jnp.sqrt(jnp.sum(dvt * dvt, axis=0, keepdims=True))
    gd = gdi[:, :_NV]
    ow = (2.0 * gd) * (2.0 * gd)
    outsideloss = _OUTSIDE_W * jnp.sum(ov * ow)

    total = (contactloss + insideloss + outsideloss + pose_prior_loss
             + hand_pose_prior_loss + angle_loss + hand_contact_loss)
    out_ref[...] = jnp.reshape(total, (1, 1))


def _assemble(minv, gdi, vnx, vny, vnz, insf, ngd, gn2, mds, wds, ids,
              mh, ih, hw, vt, ivt, bp, ip, lhp, rhp):
    args = [minv.reshape(1, _NVP), gdi.reshape(1, _NVP),
            vnx.reshape(1, _NVP), vny.reshape(1, _NVP), vnz.reshape(1, _NVP),
            insf.reshape(1, _NVP), ngd.reshape(1, _NVP), gn2.reshape(1, _NVP),
            mds.reshape(1, _DSP), wds.reshape(1, _DSP), ids.reshape(1, _DSP),
            mh.reshape(1, _HCPAD), ih.reshape(1, _HCPAD),
            hw.reshape(1, _HCPAD), vt, ivt, bp, ip, lhp, rhp]
    return pl.pallas_call(
        _assemble_body,
        out_shape=jax.ShapeDtypeStruct((1, 1), jnp.float32),
    )(*args)


@jax.jit
def _run(vertices, body_pose, left_hand_pose, right_hand_pose, init_pose,
         init_verts, geodist, hand_contact_prior_weights, ds,
         hand_contact_prior, faces, init_verts_in_contact):
    v = vertices[0]

    minv, mini, gdi = _stream_minmax(geodist, v, init_verts_in_contact)

    vp = jnp.pad(v, ((0, _NVP - _NV), (0, 0)))
    vx, vy, vz = vp[:, 0], vp[:, 1], vp[:, 2]
    fp = jnp.pad(faces.astype(jnp.int32), ((0, _NFP - _NF), (0, 0)))
    f0, f1, f2 = fp[:, 0], fp[:, 1], fp[:, 2]
    dsp = jnp.pad(ds.astype(jnp.int32), (0, _DSP - _DS),
                  constant_values=_NV)
    hcpp = jnp.pad(hand_contact_prior.astype(jnp.int32), (0, _HCPAD - _HC),
                   constant_values=_NV)

    vnx, vny, vnz = _sc_normals(vx, vy, vz, f0, f1, f2)
    (insf, ngd, gn2, mds, wds, ids, mh, ih) = _sc_gathers(
        vx, vy, vz, vnx, vny, vnz, mini, minv, gdi, dsp, hcpp)

    hw = jnp.pad(hand_contact_prior_weights, (0, _HCPAD - _HC))
    vt = v.T
    ivt = init_verts[0].T
    out = _assemble(minv, gdi, vnx, vny, vnz, insf, ngd, gn2, mds, wds,
                    ids, mh, ih, hw, vt, ivt, body_pose, init_pose,
                    left_hand_pose, right_hand_pose)
    return out[0, 0]


def kernel(vertices, body_pose, left_hand_pose, right_hand_pose, init_pose,
           init_verts, geodist, hand_contact_prior_weights, ds,
           hand_contact_prior, faces, init_verts_in_contact):
    return _run(vertices, body_pose, left_hand_pose, right_hand_pose,
                init_pose, init_verts, geodist, hand_contact_prior_weights,
                ds, hand_contact_prior, faces, init_verts_in_contact)


# speedup vs baseline: 9.4718x; 1.0076x over previous
import functools
import jax
import jax.numpy as jnp
from jax import lax
from jax.experimental import pallas as pl
from jax.experimental.pallas import tpu as pltpu
from jax.experimental.pallas import tpu_sc as plsc

_NV = 6890
_GEO_THRESH = 0.1
_A1, _A2, _B1, _B2 = 0.04, 0.04, 0.07, 0.06
_C1, _C2, _D1, _D2 = 0.01, 0.01, 0.023, 0.02
_INSIDE_W, _OUTSIDE_W, _CONTACT_W = 0.5, 2.0, 0.5
_HCP_W, _POSE_W, _HAND_POSE_W, _ANGLE_W = 1.0, 0.04, 0.04, 0.01

_R = 896
_NBLK = (_NV + _R - 1) // _R
_NVP = _NBLK * _R

_NF = 13776
_NFP = 14336
_HNF = _NFP // 2
_DS = 1722
_DSP = 1728
_HC = 1556
_HCPAD = 1568


def _stream_body(geo_ref, va_ref, vta_ref, ivc_ref, minv_ref, mini_ref,
                 gdi_ref, amask_ref):
    i = pl.program_id(0)

    @pl.when(i == 0)
    def _():
        col = lax.broadcasted_iota(jnp.int32, (1, _NV), 1)
        acc = jnp.full((1, _NV), 1e10, jnp.float32)
        nc = ivc_ref.shape[0]
        chunk = 100
        for j in range(0, nc, chunk):
            ids = ivc_ref[j:j + chunk, :]
            hit = jnp.max((ids == col).astype(jnp.float32), axis=0,
                          keepdims=True)
            acc = jnp.where(hit > 0.0, 0.0, acc)
        amask_ref[...] = acc

    geo = geo_ref[...]
    va = va_ref[...]
    vta = vta_ref[...]

    d2 = jnp.dot(va, vta, preferred_element_type=jnp.float32)
    d2m = jnp.where(geo < _GEO_THRESH, 1e10, jnp.maximum(d2, 1e-12))

    minv2 = jnp.min(d2m, axis=1, keepdims=True)
    idx = jnp.argmin(d2m, axis=1).astype(jnp.int32)[:, None]
    minv = jnp.where(minv2 >= 1e10, 1e5, jnp.sqrt(minv2))

    gdi = jnp.min(geo + amask_ref[...], axis=1, keepdims=True)

    minv_ref[...] = minv
    mini_ref[...] = idx
    gdi_ref[...] = gdi


def _stream_minmax(geodist, v, init_verts_in_contact):
    sq = jnp.sum(v * v, axis=1)
    one = jnp.ones((_NV,), jnp.float32)
    zero = jnp.zeros((_NV,), jnp.float32)
    va = jnp.stack([v[:, 0], v[:, 1], v[:, 2], one, sq, zero, zero, zero],
                   axis=1)
    va = jnp.pad(va, ((0, _NVP - _NV), (0, 0)))
    vta = jnp.stack([-2.0 * v[:, 0], -2.0 * v[:, 1], -2.0 * v[:, 2], sq,
                     one, zero, zero, zero], axis=0)
    ivc = init_verts_in_contact.reshape(-1, 1).astype(jnp.int32)

    out = pl.pallas_call(
        _stream_body,
        grid=(_NBLK,),
        in_specs=[
            pl.BlockSpec((_R, _NV), lambda i: (i, 0)),
            pl.BlockSpec((_R, 8), lambda i: (i, 0)),
            pl.BlockSpec((8, _NV), lambda i: (0, 0)),
            pl.BlockSpec((ivc.shape[0], 1), lambda i: (0, 0)),
        ],
        out_specs=[
            pl.BlockSpec((_R, 1), lambda i: (i, 0)),
            pl.BlockSpec((_R, 1), lambda i: (i, 0)),
            pl.BlockSpec((_R, 1), lambda i: (i, 0)),
        ],
        out_shape=[
            jax.ShapeDtypeStruct((_NVP, 1), jnp.float32),
            jax.ShapeDtypeStruct((_NVP, 1), jnp.int32),
            jax.ShapeDtypeStruct((_NVP, 1), jnp.float32),
        ],
        scratch_shapes=[pltpu.VMEM((1, _NV), jnp.float32)],
    )(geodist, va, vta, ivc)
    minv, mini, gdi = out
    return minv.reshape(_NVP), mini.reshape(_NVP), gdi.reshape(_NVP)


_SC_MESH = plsc.VectorSubcoreMesh(core_axis_name="c", subcore_axis_name="s")
_SC_PARAMS = pltpu.CompilerParams(needs_layout_passes=False)


@functools.partial(
    pl.kernel,
    out_type=[
        jax.ShapeDtypeStruct((_NVP,), jnp.float32),
        jax.ShapeDtypeStruct((_NVP,), jnp.float32),
        jax.ShapeDtypeStruct((_NVP,), jnp.float32),
    ],
    mesh=_SC_MESH,
    scratch_types=[
        pltpu.VMEM((_NVP,), jnp.float32),
        pltpu.VMEM((_NVP,), jnp.float32),
        pltpu.VMEM((_NVP,), jnp.float32),
        pltpu.VMEM((_NVP,), jnp.float32),
        pltpu.VMEM((_NVP,), jnp.float32),
        pltpu.VMEM((_NVP,), jnp.float32),
        pltpu.VMEM((_NFP,), jnp.int32),
        pltpu.VMEM((_NFP,), jnp.int32),
        pltpu.VMEM((_NFP,), jnp.int32),
    ],
    compiler_params=_SC_PARAMS,
)
def _sc_normals(vx, vy, vz, f0, f1, f2,
                vnx_o, vny_o, vnz_o,
                vxv, vyv, vzv, axv, ayv, azv, fb0, fb1, fb2):
    cid = lax.axis_index("c")
    sid = lax.axis_index("s")

    @pl.when(jnp.logical_and(cid == 0, sid == 0))
    def _():
        pltpu.sync_copy(vx, vxv)
        pltpu.sync_copy(vy, vyv)
        pltpu.sync_copy(vz, vzv)
        pltpu.sync_copy(f0, fb0)
        pltpu.sync_copy(f1, fb1)
        pltpu.sync_copy(f2, fb2)

        zero16 = jnp.zeros((16,), jnp.float32)

        def zbody(i, c):
            for u in range(4):
                sl = pl.ds(i * 64 + u * 16, 16)
                axv[sl] = zero16
                ayv[sl] = zero16
                azv[sl] = zero16
            return c
        lax.fori_loop(0, _NVP // 64, zbody, 0)

        def fbody(i, c):
            for u in range(4):
                sl = pl.ds(i * 64 + u * 16, 16)
                i0 = fb0[sl]
                i1 = fb1[sl]
                i2 = fb2[sl]
                x0 = plsc.load_gather(vxv, [i0])
                y0 = plsc.load_gather(vyv, [i0])
                z0 = plsc.load_gather(vzv, [i0])
                x1 = plsc.load_gather(vxv, [i1])
                y1 = plsc.load_gather(vyv, [i1])
                z1 = plsc.load_gather(vzv, [i1])
                x2 = plsc.load_gather(vxv, [i2])
                y2 = plsc.load_gather(vyv, [i2])
                z2 = plsc.load_gather(vzv, [i2])
                e1x, e1y, e1z = x1 - x0, y1 - y0, z1 - z0
                e2x, e2y, e2z = x2 - x0, y2 - y0, z2 - z0
                fnx = e1y * e2z - e1z * e2y
                fny = e1z * e2x - e1x * e2z
                fnz = e1x * e2y - e1y * e2x
                for ii in (i0, i1, i2):
                    plsc.addupdate_scatter(axv, [ii], fnx)
                    plsc.addupdate_scatter(ayv, [ii], fny)
                    plsc.addupdate_scatter(azv, [ii], fnz)
            return c
        lax.fori_loop(0, _NFP // 64, fbody, 0)

        pltpu.sync_copy(axv, vnx_o)
        pltpu.sync_copy(ayv, vny_o)
        pltpu.sync_copy(azv, vnz_o)


@functools.partial(
    pl.kernel,
    out_type=[
        jax.ShapeDtypeStruct((_NVP,), jnp.float32),
        jax.ShapeDtypeStruct((_NVP,), jnp.float32),
        jax.ShapeDtypeStruct((_NVP,), jnp.float32),
        jax.ShapeDtypeStruct((_DSP,), jnp.float32),
        jax.ShapeDtypeStruct((_DSP,), jnp.float32),
        jax.ShapeDtypeStruct((_DSP,), jnp.float32),
        jax.ShapeDtypeStruct((_HCPAD,), jnp.float32),
        jax.ShapeDtypeStruct((_HCPAD,), jnp.float32),
    ],
    mesh=_SC_MESH,
    scratch_types=[
        pltpu.VMEM((_NVP,), jnp.float32),
        pltpu.VMEM((_NVP,), jnp.float32),
        pltpu.VMEM((_NVP,), jnp.float32),
        pltpu.VMEM((_NVP,), jnp.float32),
        pltpu.VMEM((_NVP,), jnp.float32),
        pltpu.VMEM((_NVP,), jnp.float32),
        pltpu.VMEM((_NVP,), jnp.int32),
        pltpu.VMEM((_NVP,), jnp.float32),
        pltpu.VMEM((_NVP,), jnp.float32),
        pltpu.VMEM((_NVP,), jnp.float32),
        pltpu.VMEM((_NVP,), jnp.float32),
        pltpu.VMEM((_NVP,), jnp.float32),
        pltpu.VMEM((_DSP,), jnp.int32),
        pltpu.VMEM((_HCPAD,), jnp.int32),
        pltpu.VMEM((_DSP,), jnp.float32),
        pltpu.VMEM((_DSP,), jnp.float32),
        pltpu.VMEM((_DSP,), jnp.float32),
        pltpu.VMEM((_HCPAD,), jnp.float32),
        pltpu.VMEM((_HCPAD,), jnp.float32),
    ],
    compiler_params=_SC_PARAMS,
)
def _sc_gathers(vx, vy, vz, vnx, vny, vnz, mini, minv, gdi, ds, hcp,
                insf_o, ngd_o, gn2_o, mds_o, wds_o, ids_o, mh_o, ih_o,
                vxv, vyv, vzv, axv, ayv, azv,
                miniv, minvv, gdiv, insv, ngdv, gn2v,
                dsv, hcpv, mdsv, wdsv, idsv, mhv, ihv):
    cid = lax.axis_index("c")
    sid = lax.axis_index("s")

    @pl.when(jnp.logical_and(cid == 0, sid == 0))
    def _():
        pltpu.sync_copy(vx, vxv)
        pltpu.sync_copy(vy, vyv)
        pltpu.sync_copy(vz, vzv)
        pltpu.sync_copy(vnx, axv)
        pltpu.sync_copy(vny, ayv)
        pltpu.sync_copy(vnz, azv)
        pltpu.sync_copy(mini, miniv)
        pltpu.sync_copy(minv, minvv)
        pltpu.sync_copy(gdi, gdiv)
        pltpu.sync_copy(ds, dsv)
        pltpu.sync_copy(hcp, hcpv)

        zero16 = jnp.zeros((16,), jnp.float32)

        def zbody(i, c):
            for u in range(4):
                insv[pl.ds(i * 64 + u * 16, 16)] = zero16
            return c
        lax.fori_loop(0, _NVP // 64, zbody, 0)

        def dbody(i, c):
            for u in range(2):
                d = dsv[pl.ds(i * 32 + u * 16, 16)]
                nr = plsc.load_gather(miniv, [d])
                nr = jnp.minimum(jnp.maximum(nr, 0), _NV - 1)
                vxd = plsc.load_gather(vxv, [d])
                vyd = plsc.load_gather(vyv, [d])
                vzd = plsc.load_gather(vzv, [d])
                vxn = plsc.load_gather(vxv, [nr])
                vyn = plsc.load_gather(vyv, [nr])
                vzn = plsc.load_gather(vzv, [nr])
                nx = plsc.load_gather(axv, [nr])
                ny = plsc.load_gather(ayv, [nr])
                nz = plsc.load_gather(azv, [nr])
                ext = (vxd - vxn) * nx + (vyd - vyn) * ny + (vzd - vzn) * nz
                flag = jnp.where(ext > 0.0, 0.0, 1.0)
                plsc.addupdate_scatter(insv, [d], flag)
            return c
        lax.fori_loop(0, _DSP // 32, dbody, 0)

        def d2body(i, c):
            for u in range(2):
                sl = pl.ds(i * 32 + u * 16, 16)
                d = dsv[sl]
                mdsv[sl] = plsc.load_gather(minvv, [d])
                wdsv[sl] = plsc.load_gather(gdiv, [d])
                idsv[sl] = plsc.load_gather(insv, [d])
            return c
        lax.fori_loop(0, _DSP // 32, d2body, 0)

        def hbody(i, c):
            for u in range(2):
                sl = pl.ds(i * 32 + u * 16, 16)
                hh = hcpv[sl]
                mhv[sl] = plsc.load_gather(minvv, [hh])
                ihv[sl] = plsc.load_gather(insv, [hh])
            return c
        lax.fori_loop(0, _HCPAD // 32, hbody, 0)

        def nbody(i, c):
            for u in range(4):
                sl = pl.ds(i * 64 + u * 16, 16)
                aix = axv[sl]
                aiy = ayv[sl]
                aiz = azv[sl]
                ii = miniv[sl]
                ii = jnp.minimum(jnp.maximum(ii, 0), _NV - 1)
                gx = plsc.load_gather(axv, [ii])
                gy = plsc.load_gather(ayv, [ii])
                gz = plsc.load_gather(azv, [ii])
                ngdv[sl] = aix * gx + aiy * gy + aiz * gz
                gn2v[sl] = gx * gx + gy * gy + gz * gz
            return c
        lax.fori_loop(0, _NVP // 64, nbody, 0)

        pltpu.sync_copy(insv, insf_o)
        pltpu.sync_copy(ngdv, ngd_o)
        pltpu.sync_copy(gn2v, gn2_o)
        pltpu.sync_copy(mdsv, mds_o)
        pltpu.sync_copy(wdsv, wds_o)
        pltpu.sync_copy(idsv, ids_o)
        pltpu.sync_copy(mhv, mh_o)
        pltpu.sync_copy(ihv, ih_o)


def _mmean(vals, mask):
    s = jnp.sum(jnp.where(mask, vals, 0.0))
    c = jnp.sum(jnp.where(mask, 1.0, 0.0))
    return jnp.where(c > 0, s / jnp.maximum(c, 1.0), 0.0)


def _assemble_body(minv_ref, gdi_ref, vnx_ref, vny_ref, vnz_ref, insf_ref,
                   ngd_ref, gn2_ref, mds_ref, wds_ref, ids_ref, mh_ref,
                   ih_ref, hw_ref, vt_ref, ivt_ref, bp_ref, ip_ref, lhp_ref,
                   rhp_ref, out_ref):
    nvi = lax.broadcasted_iota(jnp.int32, (1, _NVP), 1)
    valid_v = nvi < _NV

    minv = minv_ref[...]
    gdi = gdi_ref[...]

    dsi = lax.broadcasted_iota(jnp.int32, (1, _DSP), 1)
    valid_ds = dsi < _DS
    inside_ds = ids_ref[...] > 0.0
    w_out = 1.0 / (5.0 * wds_ref[...] + 1.0)
    vals_out = _A1 * w_out * jnp.tanh(mds_ref[...] / _A2)
    contactloss = _CONTACT_W * _mmean(
        vals_out, jnp.logical_and(jnp.logical_not(inside_ds), valid_ds))

    inside_v = insf_ref[...] > 0.0
    vals_in = _B1 * jnp.tanh(minv / _B2)
    insideloss = _INSIDE_W * _mmean(vals_in, jnp.logical_and(inside_v, valid_v))

    hci = lax.broadcasted_iota(jnp.int32, (1, _HCPAD), 1)
    left = hci < _HC // 2
    right = jnp.logical_and(hci >= _HC // 2, hci < _HC)
    hvi = ih_ref[...] > 0.0
    not_hvi = jnp.logical_not(hvi)
    mh = mh_ref[...]
    wv = -0.1 * hw_ref[...] + 1.0
    vals_o = _C1 * jnp.tanh(mh / _C2)
    vals_i = _D1 * jnp.tanh(mh / _D2)
    hclo = (_mmean(wv * vals_o, jnp.logical_and(not_hvi, left)) +
            _mmean(wv * vals_o, jnp.logical_and(not_hvi, right)))
    hcli = (_mmean(vals_i, jnp.logical_and(hvi, left)) +
            _mmean(vals_i, jnp.logical_and(hvi, right)))
    hand_contact_loss = _HCP_W * (hcli + hclo)

    n2 = (vnx_ref[...] * vnx_ref[...] + vny_ref[...] * vny_ref[...] +
          vnz_ref[...] * vnz_ref[...])
    denom = (jnp.sqrt(n2) + 1e-8) * (jnp.sqrt(gn2_ref[...]) + 1e-8)
    ng = ngd_ref[...] / denom
    angle_loss = _ANGLE_W * _mmean(
        1.0 + ng, jnp.logical_and(minv < 0.01, valid_v))

    bp = bp_ref[...]
    ip = ip_ref[...]
    pose_prior_loss = _POSE_W * jnp.sum((bp - ip) * (bp - ip))
    lhp = lhp_ref[...]
    rhp = rhp_ref[...]
    hand_pose_prior_loss = _HAND_POSE_W * (jnp.sum(lhp * lhp) +
                                           jnp.sum(rhp * rhp))

    dvt = ivt_ref[...] - vt_ref[...]
    ov = jnp.sqrt(jnp.sum(dvt * dvt, axis=0, keepdims=True))
    gd = gdi[:, :_NV]
    ow = (2.0 * gd) * (2.0 * gd)
    outsideloss = _OUTSIDE_W * jnp.sum(ov * ow)

    total = (contactloss + insideloss + outsideloss + pose_prior_loss
             + hand_pose_prior_loss + angle_loss + hand_contact_loss)
    out_ref[...] = jnp.reshape(total, (1, 1))


def _assemble(minv, gdi, vnx, vny, vnz, insf, ngd, gn2, mds, wds, ids,
              mh, ih, hw, vt, ivt, bp, ip, lhp, rhp):
    args = [minv.reshape(1, _NVP), gdi.reshape(1, _NVP),
            vnx.reshape(1, _NVP), vny.reshape(1, _NVP), vnz.reshape(1, _NVP),
            insf.reshape(1, _NVP), ngd.reshape(1, _NVP), gn2.reshape(1, _NVP),
            mds.reshape(1, _DSP), wds.reshape(1, _DSP), ids.reshape(1, _DSP),
            mh.reshape(1, _HCPAD), ih.reshape(1, _HCPAD),
            hw.reshape(1, _HCPAD), vt, ivt, bp, ip, lhp, rhp]
    return pl.pallas_call(
        _assemble_body,
        out_shape=jax.ShapeDtypeStruct((1, 1), jnp.float32),
    )(*args)


@jax.jit
def _run(vertices, body_pose, left_hand_pose, right_hand_pose, init_pose,
         init_verts, geodist, hand_contact_prior_weights, ds,
         hand_contact_prior, faces, init_verts_in_contact):
    v = vertices[0]

    vp = jnp.pad(v, ((0, _NVP - _NV), (0, 0)))
    vx, vy, vz = vp[:, 0], vp[:, 1], vp[:, 2]
    fp = jnp.pad(faces.astype(jnp.int32), ((0, _NFP - _NF), (0, 0)))
    f0, f1, f2 = fp[:, 0], fp[:, 1], fp[:, 2]
    dsp = jnp.pad(ds.astype(jnp.int32), (0, _DSP - _DS),
                  constant_values=_NV)
    hcpp = jnp.pad(hand_contact_prior.astype(jnp.int32), (0, _HCPAD - _HC),
                   constant_values=_NV)

    vnx, vny, vnz = _sc_normals(vx, vy, vz, f0, f1, f2)

    minv, mini, gdi = _stream_minmax(geodist, v, init_verts_in_contact)
    (insf, ngd, gn2, mds, wds, ids, mh, ih) = _sc_gathers(
        vx, vy, vz, vnx, vny, vnz, mini, minv, gdi, dsp, hcpp)

    hw = jnp.pad(hand_contact_prior_weights, (0, _HCPAD - _HC))
    vt = v.T
    ivt = init_verts[0].T
    out = _assemble(minv, gdi, vnx, vny, vnz, insf, ngd, gn2, mds, wds,
                    ids, mh, ih, hw, vt, ivt, body_pose, init_pose,
                    left_hand_pose, right_hand_pose)
    return out[0, 0]


def kernel(vertices, body_pose, left_hand_pose, right_hand_pose, init_pose,
           init_verts, geodist, hand_contact_prior_weights, ds,
           hand_contact_prior, faces, init_verts_in_contact):
    return _run(vertices, body_pose, left_hand_pose, right_hand_pose,
                init_pose, init_verts, geodist, hand_contact_prior_weights,
                ds, hand_contact_prior, faces, init_verts_in_contact)
